# baseline (speedup 1.0000x reference)
import functools

import jax
import jax.numpy as jnp
from jax import lax
from jax.experimental import pallas as pl
from jax.experimental.pallas import tpu as pltpu
from jax.experimental.pallas import tpu_sc as plsc

D_MODEL = 768
SEQ_LEN = 4096
BATCH = 4
NUM_CORES = 2
NUM_SUBCORES = 16
NUM_WORKERS = NUM_CORES * NUM_SUBCORES
L_PER_WORKER = SEQ_LEN // NUM_WORKERS
CHUNK = 64
LANES = 16


def _emb_body(x_hbm, pe_hbm, table_hbm, out_hbm, idx_v, pe_v, rows_v, sem):
    cid = lax.axis_index("c")
    sid = lax.axis_index("s")
    wid = sid * NUM_CORES + cid
    l_base = wid * L_PER_WORKER

    @pl.loop(0, L_PER_WORKER // CHUNK)
    def _chunk_loop(c):
        l0 = l_base + c * CHUNK
        pltpu.sync_copy(pe_hbm.at[pl.ds(l0, CHUNK)], pe_v)

        @pl.loop(0, BATCH)
        def _batch_loop(b):
            row0 = b * SEQ_LEN + l0
            pltpu.sync_copy(x_hbm.at[pl.ds(row0, CHUNK)], idx_v)
            pltpu.async_copy(table_hbm.at[idx_v], rows_v, sem).wait()

            @pl.loop(0, CHUNK)
            def _row_loop(i):
                for j in range(0, D_MODEL, LANES):
                    plsc.addupdate(
                        rows_v.at[i, pl.ds(j, LANES)],
                        pe_v[i, pl.ds(j, LANES)],
                    )

            pltpu.sync_copy(rows_v, out_hbm.at[pl.ds(row0, CHUNK)])


@jax.jit
def _emb(x_flat, pe, table):
    mesh = plsc.VectorSubcoreMesh(core_axis_name="c", subcore_axis_name="s")
    k = pl.kernel(
        _emb_body,
        out_type=jax.ShapeDtypeStruct((BATCH * SEQ_LEN, D_MODEL), jnp.float32),
        mesh=mesh,
        scratch_types=[
            pltpu.VMEM((CHUNK,), jnp.int32),
            pltpu.VMEM((CHUNK, D_MODEL), jnp.float32),
            pltpu.VMEM((CHUNK, D_MODEL), jnp.float32),
            pltpu.SemaphoreType.DMA,
        ],
    )
    return k(x_flat, pe, table)


def kernel(x, table, pe):
    x_flat = x.reshape(-1).astype(jnp.int32)
    out = _emb(x_flat, pe, table)
    return out.reshape(x.shape[0], x.shape[1], D_MODEL)


# speedup vs baseline: 1.2201x; 1.2201x over previous
import jax
import jax.numpy as jnp
from jax import lax
from jax.experimental import pallas as pl
from jax.experimental.pallas import tpu as pltpu
from jax.experimental.pallas import tpu_sc as plsc

D_MODEL = 768
SEQ_LEN = 4096
BATCH = 4
NUM_CORES = 2
NUM_SUBCORES = 16
NUM_WORKERS = NUM_CORES * NUM_SUBCORES
L_PER_WORKER = SEQ_LEN // NUM_WORKERS
CHUNK = 32
N_CHUNKS = L_PER_WORKER // CHUNK
N_STEPS = N_CHUNKS * BATCH
LANES = 16


def _emb_body(x_hbm, pe_hbm, table_hbm, out_hbm, idx_v, pe_a, pe_b, rows_a,
              rows_b, sem_g, sem_st, sem_pe, sem_idx):
    cid = lax.axis_index("c")
    sid = lax.axis_index("s")
    wid = sid * NUM_CORES + cid
    l_base = wid * L_PER_WORKER

    pe_bufs = [pe_a, pe_b]
    rows_bufs = [rows_a, rows_b]

    idx_copies = [
        pltpu.async_copy(
            x_hbm.at[pl.ds(b * SEQ_LEN + l_base, L_PER_WORKER)],
            idx_v.at[b], sem_idx)
        for b in range(BATCH)
    ]

    pe_copies = [None] * N_CHUNKS
    pe_copies[0] = pltpu.async_copy(
        pe_hbm.at[pl.ds(l_base, CHUNK)], pe_bufs[0], sem_pe)

    for c in idx_copies:
        c.wait()

    gathers = [None] * N_STEPS
    stores = [None] * N_STEPS

    def start_gather(s):
        c, b = divmod(s, BATCH)
        idx_slice = idx_v.at[b, pl.ds(c * CHUNK, CHUNK)]
        gathers[s] = pltpu.async_copy(
            table_hbm.at[idx_slice], rows_bufs[s % 2], sem_g)

    start_gather(0)

    for s in range(N_STEPS):
        c, b = divmod(s, BATCH)
        cur = rows_bufs[s % 2]
        pe_cur = pe_bufs[c % 2]
        if b == 0:
            pe_copies[c].wait()
            if c + 1 < N_CHUNKS:
                pe_copies[c + 1] = pltpu.async_copy(
                    pe_hbm.at[pl.ds(l_base + (c + 1) * CHUNK, CHUNK)],
                    pe_bufs[(c + 1) % 2], sem_pe)
        gathers[s].wait()
        if s >= 1:
            stores[s - 1].wait()
        if s + 1 < N_STEPS:
            start_gather(s + 1)

        @pl.loop(0, CHUNK)
        def _row(i, cur=cur, pe_cur=pe_cur):
            for j in range(0, D_MODEL, LANES):
                plsc.addupdate(
                    cur.at[i, pl.ds(j, LANES)],
                    pe_cur[i, pl.ds(j, LANES)],
                )

        row0 = b * SEQ_LEN + l_base + c * CHUNK
        stores[s] = pltpu.async_copy(
            cur, out_hbm.at[pl.ds(row0, CHUNK)], sem_st)

    stores[N_STEPS - 1].wait()


@jax.jit
def _emb(x_flat, pe, table):
    mesh = plsc.VectorSubcoreMesh(core_axis_name="c", subcore_axis_name="s")
    k = pl.kernel(
        _emb_body,
        out_type=jax.ShapeDtypeStruct((BATCH * SEQ_LEN, D_MODEL), jnp.float32),
        mesh=mesh,
        scratch_types=[
            pltpu.VMEM((BATCH, L_PER_WORKER), jnp.int32),
            pltpu.VMEM((CHUNK, D_MODEL), jnp.float32),
            pltpu.VMEM((CHUNK, D_MODEL), jnp.float32),
            pltpu.VMEM((CHUNK, D_MODEL), jnp.float32),
            pltpu.VMEM((CHUNK, D_MODEL), jnp.float32),
            pltpu.SemaphoreType.DMA,
            pltpu.SemaphoreType.DMA,
            pltpu.SemaphoreType.DMA,
            pltpu.SemaphoreType.DMA,
        ],
    )
    return k(x_flat, pe, table)


def kernel(x, table, pe):
    x_flat = x.reshape(-1).astype(jnp.int32)
    out = _emb(x_flat, pe, table)
    return out.reshape(x.shape[0], x.shape[1], D_MODEL)


# speedup vs baseline: 1.2994x; 1.0650x over previous
import jax
import jax.numpy as jnp
from jax import lax
from jax.experimental import pallas as pl
from jax.experimental.pallas import tpu as pltpu
from jax.experimental.pallas import tpu_sc as plsc

D_MODEL = 768
SEQ_LEN = 4096
BATCH = 4
NUM_CORES = 2
NUM_SUBCORES = 16
NUM_WORKERS = NUM_CORES * NUM_SUBCORES
L_PER_WORKER = SEQ_LEN // NUM_WORKERS
CHUNK = 32
N_CHUNKS = L_PER_WORKER // CHUNK
N_STEPS = N_CHUNKS * BATCH
LANES = 16


NBUF = 3


def _emb_body(x_hbm, pe_hbm, table_hbm, out_hbm, idx_v, pe_a, pe_b, rows_a,
              rows_b, rows_c, sg_a, sg_b, sg_c, ss_a, ss_b, ss_c, sem_pe,
              sem_idx):
    cid = lax.axis_index("c")
    sid = lax.axis_index("s")
    wid = sid * NUM_CORES + cid
    l_base = wid * L_PER_WORKER

    pe_bufs = [pe_a, pe_b]
    rows_bufs = [rows_a, rows_b, rows_c]
    sem_g = [sg_a, sg_b, sg_c]
    sem_st = [ss_a, ss_b, ss_c]

    idx_copies = [
        pltpu.async_copy(
            x_hbm.at[pl.ds(b * SEQ_LEN + l_base, L_PER_WORKER)],
            idx_v.at[b], sem_idx)
        for b in range(BATCH)
    ]

    pe_copies = [None] * N_CHUNKS
    pe_copies[0] = pltpu.async_copy(
        pe_hbm.at[pl.ds(l_base, CHUNK)], pe_bufs[0], sem_pe)

    for c in idx_copies:
        c.wait()

    gathers = [None] * N_STEPS
    stores = [None] * N_STEPS

    def start_gather(s):
        c, b = divmod(s, BATCH)
        idx_slice = idx_v.at[b, pl.ds(c * CHUNK, CHUNK)]
        gathers[s] = pltpu.async_copy(
            table_hbm.at[idx_slice], rows_bufs[s % NBUF], sem_g[s % NBUF])

    start_gather(0)
    start_gather(1)

    for s in range(N_STEPS):
        c, b = divmod(s, BATCH)
        cur = rows_bufs[s % NBUF]
        pe_cur = pe_bufs[c % 2]
        if b == 0:
            pe_copies[c].wait()
            if c + 1 < N_CHUNKS:
                pe_copies[c + 1] = pltpu.async_copy(
                    pe_hbm.at[pl.ds(l_base + (c + 1) * CHUNK, CHUNK)],
                    pe_bufs[(c + 1) % 2], sem_pe)
        gathers[s].wait()
        if s + 2 < N_STEPS:
            if s >= 1:
                stores[s - 1].wait()
            start_gather(s + 2)

        @pl.loop(0, CHUNK)
        def _row(i, cur=cur, pe_cur=pe_cur):
            for j in range(0, D_MODEL, LANES):
                plsc.addupdate(
                    cur.at[i, pl.ds(j, LANES)],
                    pe_cur[i, pl.ds(j, LANES)],
                )

        row0 = b * SEQ_LEN + l_base + c * CHUNK
        stores[s] = pltpu.async_copy(
            cur, out_hbm.at[pl.ds(row0, CHUNK)], sem_st[s % NBUF])

    stores[N_STEPS - 2].wait()
    stores[N_STEPS - 1].wait()


@jax.jit
def _emb(x_flat, pe, table):
    mesh = plsc.VectorSubcoreMesh(core_axis_name="c", subcore_axis_name="s")
    k = pl.kernel(
        _emb_body,
        out_type=jax.ShapeDtypeStruct((BATCH * SEQ_LEN, D_MODEL), jnp.float32),
        mesh=mesh,
        scratch_types=[
            pltpu.VMEM((BATCH, L_PER_WORKER), jnp.int32),
            pltpu.VMEM((CHUNK, D_MODEL), jnp.float32),
            pltpu.VMEM((CHUNK, D_MODEL), jnp.float32),
            pltpu.VMEM((CHUNK, D_MODEL), jnp.float32),
            pltpu.VMEM((CHUNK, D_MODEL), jnp.float32),
            pltpu.VMEM((CHUNK, D_MODEL), jnp.float32),
            pltpu.SemaphoreType.DMA,
            pltpu.SemaphoreType.DMA,
            pltpu.SemaphoreType.DMA,
            pltpu.SemaphoreType.DMA,
            pltpu.SemaphoreType.DMA,
            pltpu.SemaphoreType.DMA,
            pltpu.SemaphoreType.DMA,
            pltpu.SemaphoreType.DMA,
        ],
    )
    return k(x_flat, pe, table)


def kernel(x, table, pe):
    x_flat = x.reshape(-1).astype(jnp.int32)
    out = _emb(x_flat, pe, table)
    return out.reshape(x.shape[0], x.shape[1], D_MODEL)


# speedup vs baseline: 1.5285x; 1.1763x over previous
import jax
import jax.numpy as jnp
from jax import lax
from jax.experimental import pallas as pl
from jax.experimental.pallas import tpu as pltpu
from jax.experimental.pallas import tpu_sc as plsc

D_MODEL = 768
SEQ_LEN = 4096
BATCH = 4
NUM_CORES = 2
NUM_SUBCORES = 16
NUM_WORKERS = NUM_CORES * NUM_SUBCORES
L_PER_WORKER = SEQ_LEN // NUM_WORKERS
CHUNK = 8
N_STEPS = L_PER_WORKER // CHUNK
ROWS_PER_STEP = BATCH * CHUNK
NBUF = 3
LANES = 16


def _emb_body(x_hbm, pe_hbm, table_hbm, out_hbm, idx_v, pe_v, rows_v,
              sem_g, sem_st, sem_pe, sem_idx):
    cid = lax.axis_index("c")
    sid = lax.axis_index("s")
    wid = sid * NUM_CORES + cid
    l_base = wid * L_PER_WORKER

    idx_copies = [
        pltpu.async_copy(
            x_hbm.at[pl.ds(b * SEQ_LEN + l_base, L_PER_WORKER)],
            idx_v.at[b], sem_idx)
        for b in range(BATCH)
    ]

    def start_step(s):
        slot = lax.rem(s, NBUF)
        pltpu.async_copy(
            pe_hbm.at[pl.ds(l_base + s * CHUNK, CHUNK)],
            pe_v.at[pl.ds(slot * CHUNK, CHUNK)], sem_pe)
        for b in range(BATCH):
            pltpu.async_copy(
                table_hbm.at[idx_v.at[b, pl.ds(s * CHUNK, CHUNK)]],
                rows_v.at[pl.ds(slot * ROWS_PER_STEP + b * CHUNK, CHUNK)],
                sem_g)

    def wait_pe():
        pltpu.make_async_copy(
            pe_hbm.at[pl.ds(0, CHUNK)], pe_v.at[pl.ds(0, CHUNK)],
            sem_pe).wait()

    def wait_gathers():
        pltpu.make_async_copy(
            table_hbm.at[pl.ds(0, ROWS_PER_STEP)],
            rows_v.at[pl.ds(0, ROWS_PER_STEP)], sem_g).wait()

    def wait_stores():
        pltpu.make_async_copy(
            rows_v.at[pl.ds(0, ROWS_PER_STEP)],
            out_hbm.at[pl.ds(0, ROWS_PER_STEP)], sem_st).wait()

    for c in idx_copies:
        c.wait()

    start_step(0)
    start_step(1)

    @pl.loop(0, N_STEPS)
    def _step(s):
        slot = lax.rem(s, NBUF)
        row0 = slot * ROWS_PER_STEP
        pe0 = slot * CHUNK

        wait_pe()
        wait_gathers()

        @pl.when(s >= 1)
        def _():
            wait_stores()

        @pl.when(s + 2 < N_STEPS)
        def _():
            start_step(s + 2)

        @pl.loop(0, CHUNK)
        def _pos(i):
            for j0 in range(0, D_MODEL, 4 * LANES):
                regs = [
                    pe_v[pe0 + i, pl.ds(j0 + k * LANES, LANES)]
                    for k in range(4)
                ]
                for k in range(4):
                    for b in range(BATCH):
                        plsc.addupdate(
                            rows_v.at[row0 + b * CHUNK + i,
                                      pl.ds(j0 + k * LANES, LANES)],
                            regs[k])

        for b in range(BATCH):
            pltpu.async_copy(
                rows_v.at[pl.ds(row0 + b * CHUNK, CHUNK)],
                out_hbm.at[pl.ds(b * SEQ_LEN + l_base + s * CHUNK, CHUNK)],
                sem_st)

    wait_stores()


@jax.jit
def _emb(x_flat, pe, table):
    mesh = plsc.VectorSubcoreMesh(core_axis_name="c", subcore_axis_name="s")
    k = pl.kernel(
        _emb_body,
        out_type=jax.ShapeDtypeStruct((BATCH * SEQ_LEN, D_MODEL), jnp.float32),
        mesh=mesh,
        scratch_types=[
            pltpu.VMEM((BATCH, L_PER_WORKER), jnp.int32),
            pltpu.VMEM((NBUF * CHUNK, D_MODEL), jnp.float32),
            pltpu.VMEM((NBUF * ROWS_PER_STEP, D_MODEL), jnp.float32),
            pltpu.SemaphoreType.DMA,
            pltpu.SemaphoreType.DMA,
            pltpu.SemaphoreType.DMA,
            pltpu.SemaphoreType.DMA,
        ],
    )
    return k(x_flat, pe, table)


def kernel(x, table, pe):
    x_flat = x.reshape(-1).astype(jnp.int32)
    out = _emb(x_flat, pe, table)
    return out.reshape(x.shape[0], x.shape[1], D_MODEL)


# speedup vs baseline: 1.5337x; 1.0034x over previous
import jax
import jax.numpy as jnp
from jax import lax
from jax.experimental import pallas as pl
from jax.experimental.pallas import tpu as pltpu
from jax.experimental.pallas import tpu_sc as plsc

D_MODEL = 768
SEQ_LEN = 4096
BATCH = 4
NUM_CORES = 2
NUM_SUBCORES = 16
NUM_WORKERS = NUM_CORES * NUM_SUBCORES
L_PER_WORKER = SEQ_LEN // NUM_WORKERS
CHUNK = 8
N_STEPS = L_PER_WORKER // CHUNK
ROWS_PER_STEP = BATCH * CHUNK
NBUF = 4
LANES = 16


def _emb_body(x_hbm, pe_hbm, table_hbm, out_hbm, idx_v, pe_v, rows_v,
              sem_g, sem_st, sem_pe, sem_idx):
    cid = lax.axis_index("c")
    sid = lax.axis_index("s")
    wid = sid * NUM_CORES + cid
    l_base = wid * L_PER_WORKER

    idx_copies = [
        pltpu.async_copy(
            x_hbm.at[pl.ds(b * SEQ_LEN + l_base, L_PER_WORKER)],
            idx_v.at[b], sem_idx)
        for b in range(BATCH)
    ]

    def start_step(s):
        slot = lax.rem(s, NBUF)
        pltpu.async_copy(
            pe_hbm.at[pl.ds(l_base + s * CHUNK, CHUNK)],
            pe_v.at[pl.ds(slot * CHUNK, CHUNK)], sem_pe)
        for b in range(BATCH):
            pltpu.async_copy(
                table_hbm.at[idx_v.at[b, pl.ds(s * CHUNK, CHUNK)]],
                rows_v.at[pl.ds(slot * ROWS_PER_STEP + b * CHUNK, CHUNK)],
                sem_g)

    def wait_pe():
        pltpu.make_async_copy(
            pe_hbm.at[pl.ds(0, CHUNK)], pe_v.at[pl.ds(0, CHUNK)],
            sem_pe).wait()

    def wait_gathers():
        pltpu.make_async_copy(
            table_hbm.at[pl.ds(0, ROWS_PER_STEP)],
            rows_v.at[pl.ds(0, ROWS_PER_STEP)], sem_g).wait()

    def wait_stores():
        pltpu.make_async_copy(
            rows_v.at[pl.ds(0, ROWS_PER_STEP)],
            out_hbm.at[pl.ds(0, ROWS_PER_STEP)], sem_st).wait()

    for c in idx_copies:
        c.wait()

    start_step(0)
    start_step(1)

    @pl.loop(0, N_STEPS)
    def _step(s):
        slot = lax.rem(s, NBUF)
        row0 = slot * ROWS_PER_STEP
        pe0 = slot * CHUNK

        wait_pe()
        wait_gathers()

        @pl.when(s >= 2)
        def _():
            wait_stores()

        @pl.when(s + 2 < N_STEPS)
        def _():
            start_step(s + 2)

        @pl.loop(0, CHUNK)
        def _pos(i):
            for j0 in range(0, D_MODEL, 4 * LANES):
                regs = [
                    pe_v[pe0 + i, pl.ds(j0 + k * LANES, LANES)]
                    for k in range(4)
                ]
                for k in range(4):
                    for b in range(BATCH):
                        plsc.addupdate(
                            rows_v.at[row0 + b * CHUNK + i,
                                      pl.ds(j0 + k * LANES, LANES)],
                            regs[k])

        for b in range(BATCH):
            pltpu.async_copy(
                rows_v.at[pl.ds(row0 + b * CHUNK, CHUNK)],
                out_hbm.at[pl.ds(b * SEQ_LEN + l_base + s * CHUNK, CHUNK)],
                sem_st)

    wait_stores()
    wait_stores()


@jax.jit
def _emb(x_flat, pe, table):
    mesh = plsc.VectorSubcoreMesh(core_axis_name="c", subcore_axis_name="s")
    k = pl.kernel(
        _emb_body,
        out_type=jax.ShapeDtypeStruct((BATCH * SEQ_LEN, D_MODEL), jnp.float32),
        mesh=mesh,
        scratch_types=[
            pltpu.VMEM((BATCH, L_PER_WORKER), jnp.int32),
            pltpu.VMEM((NBUF * CHUNK, D_MODEL), jnp.float32),
            pltpu.VMEM((NBUF * ROWS_PER_STEP, D_MODEL), jnp.float32),
            pltpu.SemaphoreType.DMA,
            pltpu.SemaphoreType.DMA,
            pltpu.SemaphoreType.DMA,
            pltpu.SemaphoreType.DMA,
        ],
    )
    return k(x_flat, pe, table)


def kernel(x, table, pe):
    x_flat = x.reshape(-1).astype(jnp.int32)
    out = _emb(x_flat, pe, table)
    return out.reshape(x.shape[0], x.shape[1], D_MODEL)
